# Initial kernel scaffold; baseline (speedup 1.0000x reference)
import jax
import jax.numpy as jnp
from jax.experimental import pallas as pl


def kernel(memory_tensor, last_update, unique_nodes, unique_messages, unique_ts, W_ih, W_hh, b_ih, b_hh):
    raise NotImplementedError("write your pallas kernel here")


# speedup vs baseline: 1.3123x; 1.3123x over previous
import functools

import jax
import jax.numpy as jnp
from jax import lax
from jax.experimental import pallas as pl
from jax.experimental.pallas import tpu as pltpu
from jax.experimental.pallas import tpu_sc as plsc

N_NODES = 100000
MEM_DIM = 128
MSG_DIM = 256
B = 4096

_NC = 2
_NS = 16
_NW = _NC * _NS
_CHUNK = B // _NW


def _sc_mesh():
    return plsc.VectorSubcoreMesh(
        core_axis_name="c", subcore_axis_name="s", num_cores=_NC, num_subcores=_NS
    )


def _worker_id():
    return lax.axis_index("s") * _NC + lax.axis_index("c")


def _sc_gather(mem, idx):
    @functools.partial(
        pl.kernel,
        out_type=jax.ShapeDtypeStruct((B, MEM_DIM), jnp.float32),
        mesh=_sc_mesh(),
        scratch_types=[
            pltpu.VMEM((_CHUNK,), jnp.int32),
            pltpu.VMEM((_CHUNK, MEM_DIM), jnp.float32),
            pltpu.SemaphoreType.DMA,
        ],
    )
    def gk(mem_hbm, idx_hbm, out_hbm, idx_v, rows_v, sem):
        base = _worker_id() * _CHUNK
        pltpu.sync_copy(idx_hbm.at[pl.ds(base, _CHUNK)], idx_v)
        pltpu.async_copy(mem_hbm.at[idx_v], rows_v, sem).wait()
        pltpu.sync_copy(rows_v, out_hbm.at[pl.ds(base, _CHUNK)])

    return gk(mem, idx)


_GRU_BLK = 512


def _gru_body(x_ref, h_ref, wih_ref, whh_ref, bih_ref, bhh_ref, out_ref):
    x = x_ref[...].astype(jnp.bfloat16)
    h32 = h_ref[...]
    h = h32.astype(jnp.bfloat16)
    gi = jnp.dot(x, wih_ref[...], preferred_element_type=jnp.float32) + bih_ref[...]
    gh = jnp.dot(h, whh_ref[...], preferred_element_type=jnp.float32) + bhh_ref[...]
    i_r, i_z, i_n = gi[:, :MEM_DIM], gi[:, MEM_DIM : 2 * MEM_DIM], gi[:, 2 * MEM_DIM :]
    h_r, h_z, h_n = gh[:, :MEM_DIM], gh[:, MEM_DIM : 2 * MEM_DIM], gh[:, 2 * MEM_DIM :]
    r = jax.nn.sigmoid(i_r + h_r)
    z = jax.nn.sigmoid(i_z + h_z)
    n = jnp.tanh(i_n + r * h_n)
    out_ref[...] = n + z * (h32 - n)


def _tc_gru(x, h, W_ih, W_hh, b_ih, b_hh):
    wih_t = W_ih.T.astype(jnp.bfloat16)
    whh_t = W_hh.T.astype(jnp.bfloat16)
    bih = b_ih.reshape(1, -1)
    bhh = b_hh.reshape(1, -1)
    grid = B // _GRU_BLK
    return pl.pallas_call(
        _gru_body,
        grid=(grid,),
        in_specs=[
            pl.BlockSpec((_GRU_BLK, MSG_DIM), lambda i: (i, 0)),
            pl.BlockSpec((_GRU_BLK, MEM_DIM), lambda i: (i, 0)),
            pl.BlockSpec((MSG_DIM, 3 * MEM_DIM), lambda i: (0, 0)),
            pl.BlockSpec((MEM_DIM, 3 * MEM_DIM), lambda i: (0, 0)),
            pl.BlockSpec((1, 3 * MEM_DIM), lambda i: (0, 0)),
            pl.BlockSpec((1, 3 * MEM_DIM), lambda i: (0, 0)),
        ],
        out_specs=pl.BlockSpec((_GRU_BLK, MEM_DIM), lambda i: (i, 0)),
        out_shape=jax.ShapeDtypeStruct((B, MEM_DIM), jnp.float32),
    )(x, h, wih_t, whh_t, bih, bhh)


_JL_CHUNK = 512


def _jlast_body(nlane_ref, nbcast_ref, out_ref):
    ni = nlane_ref[0]
    best = jnp.full((1, 128), -1, jnp.int32)
    for c in range(B // _JL_CHUNK):
        nj = nbcast_ref[pl.ds(c * _JL_CHUNK, _JL_CHUNK), :]
        jv = lax.broadcasted_iota(jnp.int32, (_JL_CHUNK, 128), 0) + c * _JL_CHUNK
        m = jnp.where(nj == ni, jv, -1)
        best = jnp.maximum(best, jnp.max(m, axis=0, keepdims=True))
    out_ref[0] = best


def _tc_jlast(nodes):
    nlane = nodes.reshape(B // 128, 1, 128)
    nbcast = jnp.broadcast_to(nodes.reshape(B, 1), (B, 128))
    out = pl.pallas_call(
        _jlast_body,
        grid=(B // 128,),
        in_specs=[
            pl.BlockSpec((1, 1, 128), lambda i: (i, 0, 0)),
            pl.BlockSpec((B, 128), lambda i: (0, 0)),
        ],
        out_specs=pl.BlockSpec((1, 1, 128), lambda i: (i, 0, 0)),
        out_shape=jax.ShapeDtypeStruct((B // 128, 1, 128), jnp.int32),
    )(nlane, nbcast)
    return out.reshape(B)


def _sc_scatter(new_h, j_last, idx, ts, mem_ref, lu_ref):
    @functools.partial(
        pl.kernel,
        out_type=(),
        mesh=_sc_mesh(),
        scratch_types=[
            pltpu.VMEM((_CHUNK,), jnp.int32),
            pltpu.VMEM((_CHUNK,), jnp.int32),
            pltpu.VMEM((_CHUNK, MEM_DIM), jnp.float32),
            pltpu.VMEM((_CHUNK,), jnp.float32),
            pltpu.SemaphoreType.DMA,
        ],
    )
    def sk(newh_hbm, jl_hbm, idx_hbm, ts_hbm, outmem_hbm, outlu_hbm, jl_v, idx_v, rows_v, ts_v, sem):
        base = _worker_id() * _CHUNK
        pltpu.sync_copy(jl_hbm.at[pl.ds(base, _CHUNK)], jl_v)
        pltpu.sync_copy(idx_hbm.at[pl.ds(base, _CHUNK)], idx_v)
        pltpu.async_copy(newh_hbm.at[jl_v], rows_v, sem).wait()
        pltpu.async_copy(ts_hbm.at[jl_v], ts_v, sem).wait()
        pltpu.async_copy(rows_v, outmem_hbm.at[idx_v], sem).wait()
        pltpu.async_copy(ts_v, outlu_hbm.at[idx_v], sem).wait()

    sk(new_h, j_last, idx, ts, mem_ref, lu_ref)


def kernel(memory_tensor, last_update, unique_nodes, unique_messages, unique_ts, W_ih, W_hh, b_ih, b_hh):
    h = _sc_gather(memory_tensor, unique_nodes)
    new_h = _tc_gru(unique_messages, h, W_ih, W_hh, b_ih, b_hh)
    j_last = _tc_jlast(unique_nodes)
    mem_ref = jax.new_ref(memory_tensor)
    lu_ref = jax.new_ref(last_update)
    _sc_scatter(new_h, j_last, unique_nodes, unique_ts, mem_ref, lu_ref)
    return mem_ref[...], lu_ref[...]
